# baseline (speedup 1.0000x reference)
import functools

import jax
import jax.numpy as jnp
from jax import lax
from jax.experimental import pallas as pl
from jax.experimental.pallas import tpu as pltpu
from jax.experimental.pallas import tpu_sc as plsc

NC = 2
NS = 16
L = 16
NW = NC * NS


def _relmm_body(feat_ref, w_ref, y_ref):
    y_ref[0] = jnp.dot(feat_ref[...], w_ref[0],
                       preferred_element_type=jnp.float32)


def _rel_matmul(feat, weight, bn):
    n, d = feat.shape
    r = weight.shape[0]
    return pl.pallas_call(
        _relmm_body,
        grid=(n // bn, r),
        in_specs=[
            pl.BlockSpec((bn, d), lambda nb, rb: (nb, 0)),
            pl.BlockSpec((1, d, d), lambda nb, rb: (rb, 0, 0)),
        ],
        out_specs=pl.BlockSpec((1, bn, d), lambda nb, rb: (rb, nb, 0)),
        out_shape=jax.ShapeDtypeStruct((r, n, d), jnp.float32),
    )(feat, weight)


def _add_body(a_ref, b_ref, o_ref):
    o_ref[...] = a_ref[...] + b_ref[...]


def _add_halves(a, b, bn):
    n, d = a.shape
    return pl.pallas_call(
        _add_body,
        grid=(n // bn,),
        in_specs=[
            pl.BlockSpec((bn, d), lambda i: (i, 0)),
            pl.BlockSpec((bn, d), lambda i: (i, 0)),
        ],
        out_specs=pl.BlockSpec((bn, d), lambda i: (i, 0)),
        out_shape=jax.ShapeDtypeStruct((n, d), jnp.float32),
    )(a, b)


def _make_sc_gather_scatter(n, d, ep, c, sec):
    ew = ep // NW
    nchunk = ew // c
    nsec = nchunk // sec
    npair = sec // 2
    rt = ((-(-n // NS) + 7) // 8) * 8
    npad = rt * NS

    mesh = plsc.VectorSubcoreMesh(core_axis_name="c", subcore_axis_name="s",
                                  num_cores=NC, num_subcores=NS)

    @functools.partial(
        pl.kernel,
        out_type=jax.ShapeDtypeStruct((2, npad, d), jnp.float32),
        mesh=mesh,
        scratch_types=[
            pltpu.VMEM((sec, c), jnp.int32),
            pltpu.VMEM((sec, c), jnp.int32),
            pltpu.VMEM((c, d), jnp.float32),
            pltpu.VMEM((c, d), jnp.float32),
            pltpu.VMEM_SHARED((npad, d), jnp.float32),
            pltpu.SemaphoreType.DMA,
            pltpu.SemaphoreType.DMA,
            pltpu.SemaphoreType.DMA,
            pltpu.SemaphoreType.DMA,
        ],
    )
    def sc_kernel(y_hbm, et_hbm, src_hbm, dst_hbm, zeros_hbm, out_hbm,
                  key_v, dst_v, rows_a, rows_b, acc,
                  gsem_a, gsem_b, ssem_a, ssem_b):
        ci = lax.axis_index("c")
        si = lax.axis_index("s")
        wid = ci * NS + si

        pltpu.sync_copy(zeros_hbm, acc.at[pl.ds(si * rt, rt)])
        plsc.subcore_barrier()

        def sec_body(s, _):
            row0 = s * sec
            pltpu.sync_copy(et_hbm.at[wid, pl.ds(row0, sec)], key_v)
            pltpu.sync_copy(src_hbm.at[wid, pl.ds(row0, sec)], dst_v)

            def key_body(i, _):
                for j in range(c // L):
                    sl = pl.ds(j * L, L)
                    key_v[i, sl] = key_v[i, sl] * n + dst_v[i, sl]
                return 0

            lax.fori_loop(0, sec, key_body, 0)
            pltpu.sync_copy(dst_hbm.at[wid, pl.ds(row0, sec)], dst_v)

            def chunk_body(i, _):
                pltpu.async_copy(y_hbm.at[key_v.at[i]], rows_a, gsem_a).wait()
                pltpu.sync_copy(rows_a, acc.at[dst_v.at[i]], add=True)
                return 0

            lax.fori_loop(0, 2 * npair, chunk_body, 0)
            return 0

        lax.fori_loop(0, nsec, sec_body, 0)

        plsc.subcore_barrier()
        pltpu.sync_copy(acc.at[pl.ds(si * rt, rt)],
                        out_hbm.at[ci, pl.ds(si * rt, rt)])

    return sc_kernel


def kernel(feat, edge_index, etypes, E_per_rel, weight):
    n, d = feat.shape
    r = weight.shape[0]
    e = etypes.shape[0]

    y = _rel_matmul(feat, weight, bn=1000).reshape(r * n, d)

    c = 128
    sec = 40
    rt = ((-(-n // NS) + 7) // 8) * 8
    npad = rt * NS
    ewp = -(-e // (NW * c * sec)) * c * sec
    ep = NW * ewp
    src = jnp.pad(edge_index[0], (0, ep - e)).reshape(NW, ewp // c, c)
    dst = jnp.pad(edge_index[1], (0, ep - e),
                  constant_values=npad - 1).reshape(NW, ewp // c, c)
    et3 = jnp.pad(etypes, (0, ep - e)).reshape(NW, ewp // c, c)
    zeros = jnp.zeros((rt, d), jnp.float32)

    sc = _make_sc_gather_scatter(n, d, ep, c=c, sec=sec)
    partials = sc(y, et3, src, dst, zeros)

    return _add_halves(partials[0, :n], partials[1, :n], bn=1000)


# speedup vs baseline: 1.7181x; 1.7181x over previous
import functools

import jax
import jax.numpy as jnp
from jax import lax
from jax.experimental import pallas as pl
from jax.experimental.pallas import tpu as pltpu
from jax.experimental.pallas import tpu_sc as plsc

NC = 2
NS = 16
L = 16
NW = NC * NS


def _relmm_body(feat_ref, w_ref, y_ref):
    y_ref[0] = jnp.dot(feat_ref[...], w_ref[0],
                       preferred_element_type=jnp.float32)


def _rel_matmul(feat, weight, bn):
    n, d = feat.shape
    r = weight.shape[0]
    return pl.pallas_call(
        _relmm_body,
        grid=(n // bn, r),
        in_specs=[
            pl.BlockSpec((bn, d), lambda nb, rb: (nb, 0)),
            pl.BlockSpec((1, d, d), lambda nb, rb: (rb, 0, 0)),
        ],
        out_specs=pl.BlockSpec((1, bn, d), lambda nb, rb: (rb, nb, 0)),
        out_shape=jax.ShapeDtypeStruct((r, n, d), jnp.float32),
    )(feat, weight)


def _add_body(a_ref, b_ref, o_ref):
    o_ref[...] = a_ref[...] + b_ref[...]


def _add_halves(a, b, bn):
    n, d = a.shape
    return pl.pallas_call(
        _add_body,
        grid=(n // bn,),
        in_specs=[
            pl.BlockSpec((bn, d), lambda i: (i, 0)),
            pl.BlockSpec((bn, d), lambda i: (i, 0)),
        ],
        out_specs=pl.BlockSpec((bn, d), lambda i: (i, 0)),
        out_shape=jax.ShapeDtypeStruct((n, d), jnp.float32),
    )(a, b)


def _make_sc_gather_scatter(n, d, ep, c, sec):
    ew = ep // NW
    nchunk = ew // c
    nsec = nchunk // sec
    npair = sec // 2
    rt = ((-(-n // NS) + 7) // 8) * 8
    npad = rt * NS

    mesh = plsc.VectorSubcoreMesh(core_axis_name="c", subcore_axis_name="s",
                                  num_cores=NC, num_subcores=NS)

    @functools.partial(
        pl.kernel,
        out_type=jax.ShapeDtypeStruct((2, npad, d), jnp.float32),
        mesh=mesh,
        scratch_types=[
            pltpu.VMEM((sec, c), jnp.int32),
            pltpu.VMEM((sec, c), jnp.int32),
            pltpu.VMEM((c, d), jnp.float32),
            pltpu.VMEM((c, d), jnp.float32),
            pltpu.VMEM_SHARED((npad, d), jnp.float32),
            pltpu.SemaphoreType.DMA,
            pltpu.SemaphoreType.DMA,
            pltpu.SemaphoreType.DMA,
            pltpu.SemaphoreType.DMA,
        ],
    )
    def sc_kernel(y_hbm, et_hbm, src_hbm, dst_hbm, zeros_hbm, out_hbm,
                  key_v, dst_v, rows_a, rows_b, acc,
                  gsem_a, gsem_b, ssem_a, ssem_b):
        ci = lax.axis_index("c")
        si = lax.axis_index("s")
        wid = ci * NS + si

        pltpu.sync_copy(zeros_hbm, acc.at[pl.ds(si * rt, rt)])
        plsc.subcore_barrier()

        def sec_body(s, _):
            pltpu.sync_copy(et_hbm.at[wid, s], key_v)
            pltpu.sync_copy(src_hbm.at[wid, s], dst_v)

            def key_body(i, _):
                for j in range(c // L):
                    sl = pl.ds(j * L, L)
                    key_v[i, sl] = key_v[i, sl] * n + dst_v[i, sl]
                return 0

            lax.fori_loop(0, sec, key_body, 0)
            pltpu.sync_copy(dst_hbm.at[wid, s], dst_v)

            def chunk_body(i, _):
                pltpu.async_copy(y_hbm.at[key_v.at[i]], rows_a, gsem_a).wait()
                pltpu.sync_copy(rows_a, acc.at[dst_v.at[i]], add=True)
                return 0

            lax.fori_loop(0, 2 * npair, chunk_body, 0)
            return 0

        lax.fori_loop(0, nsec, sec_body, 0)

        plsc.subcore_barrier()
        pltpu.sync_copy(acc.at[pl.ds(si * rt, rt)],
                        out_hbm.at[ci, pl.ds(si * rt, rt)])

    return sc_kernel


def kernel(feat, edge_index, etypes, E_per_rel, weight):
    n, d = feat.shape
    r = weight.shape[0]
    e = etypes.shape[0]

    y = _rel_matmul(feat, weight, bn=1000).reshape(r * n, d)

    c = 80
    sec = 25
    rt = ((-(-n // NS) + 7) // 8) * 8
    npad = rt * NS
    ewp = -(-e // (NW * c * sec)) * c * sec
    ep = NW * ewp
    src = jnp.pad(edge_index[0], (0, ep - e)).reshape(NW, ewp // (sec * c), sec, c)
    dst = jnp.pad(edge_index[1], (0, ep - e),
                  constant_values=npad - 1).reshape(NW, ewp // (sec * c), sec, c)
    et3 = jnp.pad(etypes, (0, ep - e)).reshape(NW, ewp // (sec * c), sec, c)
    zeros = jnp.zeros((rt, d), jnp.float32)

    sc = _make_sc_gather_scatter(n, d, ep, c=c, sec=sec)
    partials = sc(y, et3, src, dst, zeros)

    return _add_halves(partials[0, :n], partials[1, :n], bn=1000)
